# baseline (speedup 1.0000x reference)
import functools

import jax
import jax.numpy as jnp
from jax import lax
from jax.experimental import pallas as pl
from jax.experimental.pallas import tpu as pltpu
from jax.experimental.pallas import tpu_sc as plsc

N = 50000
M = 50000
D = 128
PAD = 50176
NW = 32
PER_W = PAD // NW
BLKS = PER_W // 16
NEG = -9e15


_TCB = 512
_TCG = PAD // _TCB


def _tc_body(ne, ee, w1, w2, a1r, a2h, a2t, wh_o, wf_o, s1_o, s2_o, t_o):
    wh = jnp.dot(ne[...], w1[...], preferred_element_type=jnp.float32)
    wf = jnp.dot(ee[...], w2[...], preferred_element_type=jnp.float32)
    wh_o[...] = wh
    wf_o[...] = wf
    lr = jnp.where(wh > 0, wh, 0.2 * wh)
    s1_o[0, 0, :] = jnp.sum(lr * a1r[...], axis=1)
    s2_o[0, 0, :] = jnp.sum(wf * a2h[...], axis=1)
    t_o[0, 0, :] = jnp.sum(wh * a2t[...], axis=1)


def _tc_call(ne, ee, w1, w2, a1r, a2h, a2t):
    full = pl.BlockSpec((D, D), lambda i: (0, 0))
    row = pl.BlockSpec((1, D), lambda i: (0, 0))
    big = pl.BlockSpec((_TCB, D), lambda i: (i, 0))
    sca = pl.BlockSpec((1, 1, _TCB), lambda i: (i, 0, 0))
    return pl.pallas_call(
        _tc_body,
        grid=(_TCG,),
        in_specs=[big, big, full, full, row, row, row],
        out_specs=[big, big, sca, sca, sca],
        out_shape=[
            jax.ShapeDtypeStruct((N, D), jnp.float32),
            jax.ShapeDtypeStruct((M, D), jnp.float32),
            jax.ShapeDtypeStruct((_TCG, 1, _TCB), jnp.float32),
            jax.ShapeDtypeStruct((_TCG, 1, _TCB), jnp.float32),
            jax.ShapeDtypeStruct((_TCG, 1, _TCB), jnp.float32),
        ],
    )(ne, ee, w1, w2, a1r, a2h, a2t)


def _sc_pass(tbl_hbm, out_hbm, base, with_t,
             s_v, el_v, t_v, idx, rows, out_v, sems):
    def logits_att(b, idx_ref):
        off = b * (16 * 8)
        iota = lax.iota(jnp.int32, 16)
        if with_t:
            tv = t_v[pl.ds(b * 16, 16)]
        logits = []
        for k in range(8):
            ck = plsc.load_gather(el_v, [off + iota * 8 + k])
            eik = jnp.where(ck == 0, N - 1, ck - 1)
            idx_ref[pl.ds(k * 16, 16)] = eik
            lg = plsc.load_gather(s_v, [eik])
            lg = jnp.where(ck > 0, lg, NEG)
            if with_t:
                lg = lg + tv
                lg = jnp.where(lg > 0, lg, 0.2 * lg)
            logits.append(lg)
        mx = logits[0]
        for lg in logits[1:]:
            mx = jnp.maximum(mx, lg)
        ex = [jnp.exp(lg - mx) for lg in logits]
        tot = ex[0]
        for e_k in ex[1:]:
            tot = tot + e_k
        rcp = 1.0 / tot
        return tuple(e_k * rcp for e_k in ex)

    def start(buf):
        pltpu.async_copy(tbl_hbm.at[idx[buf]], rows[buf], sems[buf])

    def wait(buf):
        pltpu.make_async_copy(tbl_hbm.at[idx[buf]], rows[buf],
                              sems[buf]).wait()

    def accum_store(b, buf, att):
        rows_v = rows[buf]
        for e in range(16):
            for j in range(8):
                sl = pl.ds(j * 16, 16)
                acc = att[0][e] * rows_v[e, sl]
                for k in range(1, 8):
                    acc = acc + att[k][e] * rows_v[k * 16 + e, sl]
                out_v[e, sl] = jnp.where(acc > 0, acc, jnp.exp(acc) - 1.0)
        row0 = base + b * 16

        @pl.when(row0 < N)
        def _():
            pltpu.sync_copy(out_v, out_hbm.at[pl.ds(row0, 16)])

    att0 = logits_att(0, idx[0])
    start(0)

    def body(i, att_c):
        b = 2 * i
        att_n = logits_att(b + 1, idx[1])
        wait(0)
        start(1)
        accum_store(b, 0, att_c)
        att_nn = logits_att(b + 2, idx[0])
        wait(1)
        start(0)
        accum_store(b + 1, 1, att_n)
        return att_nn

    lax.fori_loop(0, BLKS // 2, body, att0)
    wait(0)


def _sc_call(wh, s1, el, wf, s2, t, nl):
    mesh = plsc.VectorSubcoreMesh(core_axis_name="c", subcore_axis_name="s")

    @functools.partial(
        pl.kernel,
        mesh=mesh,
        compiler_params=pltpu.CompilerParams(needs_layout_passes=False),
        out_type=(
            jax.ShapeDtypeStruct((M, D), jnp.float32),
            jax.ShapeDtypeStruct((N, D), jnp.float32),
        ),
        scratch_types=[
            pltpu.VMEM((PAD,), jnp.float32),
            pltpu.VMEM((PER_W * 8 + 128,), jnp.int32),
            pltpu.VMEM((PER_W + 16,), jnp.float32),
            pltpu.VMEM((128,), jnp.int32),
            pltpu.VMEM((128,), jnp.int32),
            pltpu.VMEM((128, D), jnp.float32),
            pltpu.VMEM((128, D), jnp.float32),
            pltpu.VMEM((16, D), jnp.float32),
            pltpu.SemaphoreType.DMA,
            pltpu.SemaphoreType.DMA,
        ],
    )
    def body(wh_hbm, s1_hbm, el_hbm, wf_hbm, s2_hbm, t_hbm, nl_hbm,
             eo_hbm, no_hbm, s_v, el_v, t_v, idx0, idx1, rows0, rows1,
             out_v, sem0, sem1):
        wid = lax.axis_index("s") * 2 + lax.axis_index("c")
        base = wid * PER_W
        zero16 = jnp.zeros((16,), jnp.int32)
        for i in range(8):
            el_v[pl.ds(PER_W * 8 + i * 16, 16)] = zero16
        idx = (idx0, idx1)
        rows = (rows0, rows1)
        sems = (sem0, sem1)
        pltpu.sync_copy(s1_hbm, s_v)
        pltpu.sync_copy(el_hbm.at[pl.ds(base * 8, PER_W * 8)],
                        el_v.at[pl.ds(0, PER_W * 8)])
        _sc_pass(wh_hbm, eo_hbm, base, False,
                 s_v, el_v, t_v, idx, rows, out_v, sems)
        pltpu.sync_copy(s2_hbm, s_v)
        pltpu.sync_copy(nl_hbm.at[pl.ds(base * 8, PER_W * 8)],
                        el_v.at[pl.ds(0, PER_W * 8)])
        pltpu.sync_copy(t_hbm.at[pl.ds(base, PER_W)],
                        t_v.at[pl.ds(0, PER_W)])
        _sc_pass(wf_hbm, no_hbm, base, True,
                 s_v, el_v, t_v, idx, rows, out_v, sems)

    return body(wh, s1, el, wf, s2, t, nl)


def kernel(node_embs, edge_embs, edge_list, node_list, W1, W2, a1, a2):
    el = jnp.pad(edge_list, ((0, PAD - M), (0, 0))).reshape(-1)
    nl = jnp.pad(node_list, ((0, PAD - N), (0, 0))).reshape(-1)
    a1r = a1[:, 0][None, :]
    a2h = a2[:D, 0][None, :]
    a2t = a2[D:, 0][None, :]
    wh, wf, s1, s2, t = _tc_call(node_embs, edge_embs, W1, W2, a1r, a2h, a2t)
    eo, no = _sc_call(wh, s1.reshape(PAD), el, wf, s2.reshape(PAD),
                      t.reshape(PAD), nl)
    return no, eo


# speedup vs baseline: 1.4833x; 1.4833x over previous
import functools

import jax
import jax.numpy as jnp
from jax import lax
from jax.experimental import pallas as pl
from jax.experimental.pallas import tpu as pltpu
from jax.experimental.pallas import tpu_sc as plsc

N = 50000
M = 50000
D = 128
PAD = 50176
NW = 32
PER_W = PAD // NW
BLKS = PER_W // 16
NEG = -9e15


_TCB = 512
_TCG = PAD // _TCB


def _tc_body(ne, ee, w1, w2, a1r, a2h, a2t, wh_o, wf_o, s1_o, s2_o, t_o):
    wh = jnp.dot(ne[...], w1[...], preferred_element_type=jnp.float32)
    wf = jnp.dot(ee[...], w2[...], preferred_element_type=jnp.float32)
    wh_o[...] = wh
    wf_o[...] = wf
    lr = jnp.where(wh > 0, wh, 0.2 * wh)
    s1_o[0, 0, :] = jnp.sum(lr * a1r[...], axis=1)
    s2_o[0, 0, :] = jnp.sum(wf * a2h[...], axis=1)
    t_o[0, 0, :] = jnp.sum(wh * a2t[...], axis=1)


def _tc_call(ne, ee, w1, w2, a1r, a2h, a2t):
    full = pl.BlockSpec((D, D), lambda i: (0, 0))
    row = pl.BlockSpec((1, D), lambda i: (0, 0))
    big = pl.BlockSpec((_TCB, D), lambda i: (i, 0))
    sca = pl.BlockSpec((1, 1, _TCB), lambda i: (i, 0, 0))
    return pl.pallas_call(
        _tc_body,
        grid=(_TCG,),
        in_specs=[big, big, full, full, row, row, row],
        out_specs=[big, big, sca, sca, sca],
        out_shape=[
            jax.ShapeDtypeStruct((N, D), jnp.float32),
            jax.ShapeDtypeStruct((M, D), jnp.float32),
            jax.ShapeDtypeStruct((_TCG, 1, _TCB), jnp.float32),
            jax.ShapeDtypeStruct((_TCG, 1, _TCB), jnp.float32),
            jax.ShapeDtypeStruct((_TCG, 1, _TCB), jnp.float32),
        ],
    )(ne, ee, w1, w2, a1r, a2h, a2t)


def _sc_pass(tbl_hbm, out_hbm, base, with_t,
             s_v, el_v, t_v, idx_v, rows_v, out_v, sem):
    def block(b, _):
        iota = lax.iota(jnp.int32, 16)
        off = b * 128
        if with_t:
            tv = t_v[pl.ds(b * 16, 16)]
        logits = []
        for k in range(8):
            ck = plsc.load_gather(el_v, [off + iota * 8 + k])
            eik = jnp.where(ck == 0, N - 1, ck - 1)
            idx_v[pl.ds(k * 16, 16)] = eik
            lg = plsc.load_gather(s_v, [eik])
            lg = jnp.where(ck > 0, lg, NEG)
            if with_t:
                lg = lg + tv
                lg = jnp.where(lg > 0, lg, 0.2 * lg)
            logits.append(lg)
        mx = logits[0]
        for lg in logits[1:]:
            mx = jnp.maximum(mx, lg)
        ex = [jnp.exp(lg - mx) for lg in logits]
        tot = ex[0]
        for e_k in ex[1:]:
            tot = tot + e_k
        rcp = 1.0 / tot
        att = [e_k * rcp for e_k in ex]
        pltpu.async_copy(tbl_hbm.at[idx_v], rows_v, sem).wait()
        for e in range(16):
            for j in range(8):
                sl = pl.ds(j * 16, 16)
                acc = att[0][e] * rows_v[e, sl]
                for k in range(1, 8):
                    acc = acc + att[k][e] * rows_v[k * 16 + e, sl]
                out_v[e, sl] = jnp.where(acc > 0, acc, jnp.exp(acc) - 1.0)
        pltpu.sync_copy(out_v, out_hbm.at[pl.ds(base + b * 16, 16)])
        return 0

    lax.fori_loop(0, BLKS, block, 0)


def _sc_call(wh, s1, el, wf, s2, t, nl):
    mesh = plsc.VectorSubcoreMesh(core_axis_name="c", subcore_axis_name="s")

    @functools.partial(
        pl.kernel,
        mesh=mesh,
        compiler_params=pltpu.CompilerParams(needs_layout_passes=False),
        out_type=(
            jax.ShapeDtypeStruct((M, D), jnp.float32),
            jax.ShapeDtypeStruct((N, D), jnp.float32),
        ),
        scratch_types=[
            pltpu.VMEM((PAD,), jnp.float32),
            pltpu.VMEM((PER_W * 8,), jnp.int32),
            pltpu.VMEM((PER_W,), jnp.float32),
            pltpu.VMEM((128,), jnp.int32),
            pltpu.VMEM((128, D), jnp.float32),
            pltpu.VMEM((16, D), jnp.float32),
            pltpu.SemaphoreType.DMA,
        ],
    )
    def body(wh_hbm, s1_hbm, el_hbm, wf_hbm, s2_hbm, t_hbm, nl_hbm,
             eo_hbm, no_hbm, s_v, el_v, t_v, idx_v, rows_v, out_v, sem):
        wid = lax.axis_index("s") * 2 + lax.axis_index("c")
        base = jnp.minimum(wid * PER_W, M - PER_W)
        pltpu.sync_copy(s1_hbm, s_v)
        pltpu.sync_copy(el_hbm.at[pl.ds(base * 8, PER_W * 8)], el_v)
        _sc_pass(wh_hbm, eo_hbm, base, False,
                 s_v, el_v, t_v, idx_v, rows_v, out_v, sem)
        pltpu.sync_copy(s2_hbm, s_v)
        pltpu.sync_copy(nl_hbm.at[pl.ds(base * 8, PER_W * 8)], el_v)
        pltpu.sync_copy(t_hbm.at[pl.ds(base, PER_W)], t_v)
        _sc_pass(wf_hbm, no_hbm, base, True,
                 s_v, el_v, t_v, idx_v, rows_v, out_v, sem)

    return body(wh, s1, el, wf, s2, t, nl)


def kernel(node_embs, edge_embs, edge_list, node_list, W1, W2, a1, a2):
    a1r = a1[:, 0][None, :]
    a2h = a2[:D, 0][None, :]
    a2t = a2[D:, 0][None, :]
    wh, wf, s1, s2, t = _tc_call(node_embs, edge_embs, W1, W2, a1r, a2h, a2t)
    el2 = edge_list.reshape(-1)
    nl2 = node_list.reshape(-1)
    eo, no = _sc_call(wh, s1.reshape(PAD), el2, wf, s2.reshape(PAD),
                      t.reshape(PAD), nl2)
    return no, eo
